# speedup vs baseline: 5.5723x; 5.5723x over previous
import functools

import jax
import jax.numpy as jnp
from jax import lax
from jax.experimental import pallas as pl
from jax.experimental.pallas import tpu as pltpu
from jax.experimental.pallas import tpu_sc as plsc

N = 10000
D = 128
E = 320000
NC, NS = 2, 16
NW = NC * NS
NPAD = 10240
RPT = NPAD // NS
C = 128

NCH_E = E // C
ROUNDS_E = (NCH_E + NW - 1) // NW
NCH_DEG = (2 * E) // C
ROUNDS_DEG = (NCH_DEG + NW - 1) // NW

TCB = 2048
TCG = NPAD // TCB


def _deg_body(edges_hbm, deg_out, idx_v, ones_v, stage_v, deg_acc):
    c = lax.axis_index("c")
    s = lax.axis_index("s")
    wid = s * NC + c
    row0 = s * RPT
    for j in range(C // 16):
        ones_v[pl.ds(j * 16, 16)] = jnp.ones((16,), jnp.float32)
    for j in range(RPT // 16):
        stage_v[pl.ds(j * 16, 16)] = jnp.zeros((16,), jnp.float32)
    pltpu.sync_copy(stage_v, deg_acc.at[pl.ds(row0, RPT)])
    plsc.subcore_barrier()

    def body(i, carry):
        g = wid + i * NW

        @pl.when(g < NCH_DEG)
        def _():
            pltpu.sync_copy(edges_hbm.at[pl.ds(g * C, C)], idx_v)
            pltpu.sync_copy(ones_v, deg_acc.at[idx_v], add=True)

        return carry

    lax.fori_loop(0, ROUNDS_DEG, body, 0)
    plsc.subcore_barrier()
    pltpu.sync_copy(deg_acc.at[pl.ds(row0, RPT)], stage_v)
    pltpu.sync_copy(stage_v, deg_out.at[c, pl.ds(row0, RPT)])


def _agg_body(h_hbm, src_hbm, dst_hbm, out_hbm, sidx, didx, rows, acc, sem):
    c = lax.axis_index("c")
    s = lax.axis_index("s")
    wid = s * NC + c
    row0 = s * RPT
    for i in range(16):
        for j in range(D // 16):
            rows[i, pl.ds(j * 16, 16)] = jnp.zeros((16,), jnp.float32)

    def zbody(k, carry):
        pltpu.sync_copy(rows.at[pl.ds(0, 16)], acc.at[pl.ds(row0 + k * 16, 16)])
        return carry

    lax.fori_loop(0, RPT // 16, zbody, 0)
    plsc.subcore_barrier()

    def body(i, carry):
        g = wid + i * NW

        @pl.when(g < NCH_E)
        def _():
            pltpu.sync_copy(src_hbm.at[pl.ds(g * C, C)], sidx)
            pltpu.sync_copy(dst_hbm.at[pl.ds(g * C, C)], didx)
            pltpu.async_copy(h_hbm.at[sidx], rows, sem).wait()
            pltpu.sync_copy(rows, acc.at[didx], add=True)

        return carry

    lax.fori_loop(0, ROUNDS_E, body, 0)
    plsc.subcore_barrier()
    for k in range(RPT // C):
        pltpu.sync_copy(acc.at[pl.ds(row0 + k * C, C)], rows)
        pltpu.sync_copy(rows, out_hbm.at[c, pl.ds(row0 + k * C, C)])


def _sc_degree(edges_flat):
    mesh = plsc.VectorSubcoreMesh(core_axis_name="c", subcore_axis_name="s")
    f = pl.kernel(
        _deg_body,
        out_type=jax.ShapeDtypeStruct((NC, NPAD), jnp.float32),
        mesh=mesh,
        scratch_types=[
            pltpu.VMEM((C,), jnp.int32),
            pltpu.VMEM((C,), jnp.float32),
            pltpu.VMEM((RPT,), jnp.float32),
            pltpu.VMEM_SHARED((NPAD,), jnp.float32),
        ],
    )
    return f(edges_flat)


def _sc_aggregate(h, src, dst):
    mesh = plsc.VectorSubcoreMesh(core_axis_name="c", subcore_axis_name="s")
    f = pl.kernel(
        _agg_body,
        out_type=jax.ShapeDtypeStruct((NC, NPAD, D), jnp.float32),
        mesh=mesh,
        scratch_types=[
            pltpu.VMEM((C,), jnp.int32),
            pltpu.VMEM((C,), jnp.int32),
            pltpu.VMEM((C, D), jnp.float32),
            pltpu.VMEM_SHARED((NPAD, D), jnp.float32),
            pltpu.SemaphoreType.DMA,
        ],
    )
    return f(h, src, dst)


def _prep_body(deg_ref, emb_ref, norm_ref, h_ref):
    d = deg_ref[0] + deg_ref[1]
    n = lax.rsqrt(jnp.maximum(d, 1.0))
    norm_ref[...] = n
    h_ref[...] = emb_ref[...] * n


def _tc_prep(deg2, emb):
    return pl.pallas_call(
        _prep_body,
        grid=(TCG,),
        in_specs=[
            pl.BlockSpec((NC, TCB, 1), lambda i: (0, i, 0)),
            pl.BlockSpec((TCB, D), lambda i: (i, 0)),
        ],
        out_specs=[
            pl.BlockSpec((TCB, 1), lambda i: (i, 0)),
            pl.BlockSpec((TCB, D), lambda i: (i, 0)),
        ],
        out_shape=[
            jax.ShapeDtypeStruct((NPAD, 1), jnp.float32),
            jax.ShapeDtypeStruct((NPAD, D), jnp.float32),
        ],
    )(deg2, emb)


def _comb_body(scale, p_ref, norm_ref, s_ref, sout_ref, hout_ref):
    nrm = norm_ref[...]
    e = (p_ref[0] + p_ref[1]) * nrm
    sout_ref[...] = (s_ref[...] + e) * scale
    hout_ref[...] = e * nrm


def _tc_combine(p, norm, s_in, scale):
    return pl.pallas_call(
        functools.partial(_comb_body, scale),
        grid=(TCG,),
        in_specs=[
            pl.BlockSpec((NC, TCB, D), lambda i: (0, i, 0)),
            pl.BlockSpec((TCB, 1), lambda i: (i, 0)),
            pl.BlockSpec((TCB, D), lambda i: (i, 0)),
        ],
        out_specs=[
            pl.BlockSpec((TCB, D), lambda i: (i, 0)),
            pl.BlockSpec((TCB, D), lambda i: (i, 0)),
        ],
        out_shape=[
            jax.ShapeDtypeStruct((NPAD, D), jnp.float32),
            jax.ShapeDtypeStruct((NPAD, D), jnp.float32),
        ],
    )(p, norm, s_in)


def kernel(entity_embedding, edge_index):
    src = edge_index[0]
    dst = edge_index[1]
    edges_flat = edge_index.reshape(2 * E)
    emb = jnp.pad(entity_embedding, ((0, NPAD - N), (0, 0)))

    deg2 = _sc_degree(edges_flat).reshape(NC, NPAD, 1)
    norm, h = _tc_prep(deg2, emb)

    s_acc = emb
    for layer in range(3):
        p = _sc_aggregate(h, src, dst)
        scale = 0.25 if layer == 2 else 1.0
        s_acc, h = _tc_combine(p, norm, s_acc, scale)

    return s_acc[:N]
